# baseline (speedup 1.0000x reference)
import functools

import jax
import jax.numpy as jnp
from jax import lax
from jax.experimental import pallas as pl
from jax.experimental.pallas import tpu as pltpu
from jax.experimental.pallas import tpu_sc as plsc

N = 512
D = 1024
E = 8
EP = 128
FF = 2048
BLK = 128
NBLK = 16
NSLOT = NBLK * BLK
EPS = 1e-5
ALPHA = 1.702
LIMIT = 7.0
NC, NS = 2, 16
NW = NC * NS


def _route_body(x_ref, scale_ref, gw_ref, gb_ref,
                t_ref, be_ref, act_ref, s0_ref, s1_ref, w0_ref, w1o_ref):
    x = x_ref[...]
    t = x * lax.rsqrt(jnp.mean(x * x, axis=1, keepdims=True) + EPS)
    t = t * scale_ref[...]
    t_ref[...] = t.astype(jnp.bfloat16)

    lane = lax.broadcasted_iota(jnp.int32, (N, EP), 1)
    g = jax.lax.dot_general(t, gw_ref[...], (((1,), (1,)), ((), ())),
                            preferred_element_type=jnp.float32)
    g = g + gb_ref[...]
    g = jnp.where(lane < E, g, -jnp.inf)

    m1 = jnp.max(g, axis=1, keepdims=True)
    i1 = jnp.min(jnp.where(g == m1, lane, EP), axis=1, keepdims=True)
    g2 = jnp.where(lane == i1, -jnp.inf, g)
    m2 = jnp.max(g2, axis=1, keepdims=True)
    i2 = jnp.min(jnp.where(g2 == m2, lane, EP), axis=1, keepdims=True)
    e2 = jnp.exp(m2 - m1)
    w1c = 1.0 / (1.0 + e2)
    w2c = e2 / (1.0 + e2)

    oh1 = (lane == i1).astype(jnp.float32)
    oh2 = (lane == i2).astype(jnp.float32)
    oh = oh1 + oh2
    counts = jnp.sum(oh, axis=0, keepdims=True)

    r_iota = lax.broadcasted_iota(jnp.int32, (N, N), 0)
    c_iota = lax.broadcasted_iota(jnp.int32, (N, N), 1)
    tril = (r_iota >= c_iota).astype(jnp.float32)
    ranks = jax.lax.dot_general(tril, oh, (((1,), (0,)), ((), ())),
                                preferred_element_type=jnp.float32)
    rank1 = jnp.sum(ranks * oh1, axis=1, keepdims=True)
    rank2 = jnp.sum(ranks * oh2, axis=1, keepdims=True)

    nb = jnp.ceil(counts * (1.0 / BLK))
    lane_r = lax.broadcasted_iota(jnp.int32, (EP, EP), 0)
    lane_c = lax.broadcasted_iota(jnp.int32, (EP, EP), 1)
    ut = (lane_r < lane_c).astype(jnp.float32)
    bs = jax.lax.dot_general(nb, ut, (((1,), (0,)), ((), ())),
                             preferred_element_type=jnp.float32)
    nbt = jnp.sum(nb, axis=1, keepdims=True)
    slot_base = bs * BLK

    sb1 = jnp.sum(slot_base * oh1, axis=1, keepdims=True)
    sb2 = jnp.sum(slot_base * oh2, axis=1, keepdims=True)
    slot0 = sb1 + rank1 - 1.0
    slot1 = sb2 + rank2 - 1.0
    s0_ref[...] = jnp.broadcast_to(slot0, (N, EP)).astype(jnp.int32)
    s1_ref[...] = jnp.broadcast_to(slot1, (N, EP)).astype(jnp.int32)
    w0_ref[...] = jnp.broadcast_to(w1c, (N, EP))
    w1o_ref[...] = jnp.broadcast_to(w2c, (N, EP))

    j_col = lax.broadcasted_iota(jnp.int32, (NBLK, EP), 0).astype(jnp.float32)
    jj = jnp.minimum(j_col, nbt - 1.0)
    e_lane = lax.broadcasted_iota(jnp.int32, (NBLK, EP), 1).astype(jnp.float32)
    inb = jnp.where((jj >= bs) & (jj < bs + nb), 1.0, 0.0)
    be = jnp.sum(inb * e_lane, axis=1, keepdims=True)
    be_ref[...] = jnp.broadcast_to(be, (NBLK, EP)).astype(jnp.int32)
    act_ref[...] = (j_col < nbt).astype(jnp.int32)


def _route_call(x2, scale2, gwp, gbp):
    return pl.pallas_call(
        _route_body,
        out_shape=(
            jax.ShapeDtypeStruct((N, D), jnp.bfloat16),
            jax.ShapeDtypeStruct((NBLK, EP), jnp.int32),
            jax.ShapeDtypeStruct((NBLK, EP), jnp.int32),
            jax.ShapeDtypeStruct((N, EP), jnp.int32),
            jax.ShapeDtypeStruct((N, EP), jnp.int32),
            jax.ShapeDtypeStruct((N, EP), jnp.float32),
            jax.ShapeDtypeStruct((N, EP), jnp.float32),
        ),
    )(x2, scale2, gwp, gbp)


FH = FF // 2


def _moe_body(be_ref, act_ref, t_ref, s0_ref, s1_ref, wc0_ref, wc1_ref,
              w1_ref, b1_ref, w2_ref, b2_ref, og_ref):
    j = pl.program_id(0)
    c = pl.program_id(1)

    @pl.when(act_ref[j] == 1)
    def _():
        sid = lax.broadcasted_iota(jnp.int32, (BLK, N), 0) + j * BLK
        oh0 = s0_ref[...] == sid
        oh1 = s1_ref[...] == sid
        oht = (oh0 | oh1).astype(jnp.bfloat16)
        rb = jax.lax.dot_general(oht, t_ref[...], (((1,), (0,)), ((), ())),
                                 preferred_element_type=jnp.float32
                                 ).astype(jnp.bfloat16)
        swc = (jax.lax.dot_general(oh0.astype(jnp.float32), wc0_ref[...],
                                   (((1,), (0,)), ((), ())),
                                   preferred_element_type=jnp.float32) +
               jax.lax.dot_general(oh1.astype(jnp.float32), wc1_ref[...],
                                   (((1,), (0,)), ((), ())),
                                   preferred_element_type=jnp.float32))
        w1b = w1_ref[0].astype(jnp.bfloat16)
        ht = jax.lax.dot_general(w1b, rb, (((1,), (1,)), ((), ())),
                                 preferred_element_type=jnp.float32)
        ht = ht + b1_ref[0]
        h2 = ht.reshape(FH, 2 * BLK)
        xg = jnp.minimum(h2[:, :BLK], LIMIT)
        xl = jnp.clip(h2[:, BLK:], -LIMIT, LIMIT)
        a = xg * (1.0 / (1.0 + jnp.exp(-ALPHA * xg))) * (xl + 1.0)
        at = jnp.transpose(a) * swc
        w2b = w2_ref[0].astype(jnp.bfloat16)
        o = jax.lax.dot_general(at.astype(jnp.bfloat16), w2b,
                                (((1,), (1,)), ((), ())),
                                preferred_element_type=jnp.float32)

        @pl.when(c == 0)
        def _():
            og_ref[...] = o + b2_ref[0] * swc

        @pl.when(c == 1)
        def _():
            og_ref[...] = og_ref[...] + o


def _moe_call(be, act, t_bf, s0row, s1row, wc0, wc1, w1r, b1p, w2, b2r):
    grid_spec = pltpu.PrefetchScalarGridSpec(
        num_scalar_prefetch=2,
        grid=(NBLK, 2),
        in_specs=[
            pl.BlockSpec((N, D), lambda j, c, be, act: (0, 0)),
            pl.BlockSpec((1, N), lambda j, c, be, act: (0, 0)),
            pl.BlockSpec((1, N), lambda j, c, be, act: (0, 0)),
            pl.BlockSpec((N, 1), lambda j, c, be, act: (0, 0)),
            pl.BlockSpec((N, 1), lambda j, c, be, act: (0, 0)),
            pl.BlockSpec((1, FF, D), lambda j, c, be, act: (be[j], c, 0)),
            pl.BlockSpec((1, FF, 1), lambda j, c, be, act: (be[j], c, 0)),
            pl.BlockSpec((1, D, FH), lambda j, c, be, act: (be[j], 0, c)),
            pl.BlockSpec((1, 1, D), lambda j, c, be, act: (be[j], 0, 0)),
        ],
        out_specs=pl.BlockSpec((BLK, D), lambda j, c, be, act: (j, 0)),
    )
    return pl.pallas_call(
        _moe_body,
        grid_spec=grid_spec,
        out_shape=jax.ShapeDtypeStruct((NSLOT, D), jnp.float32),
    )(be, act, t_bf, s0row, s1row, wc0, wc1, w1r, b1p, w2, b2r)


_T_PER_W = N // NW
@functools.lru_cache(maxsize=None)
def _sc_combine_kernel():
    mesh = plsc.VectorSubcoreMesh(core_axis_name="c", subcore_axis_name="s")

    @functools.partial(
        pl.kernel,
        out_type=jax.ShapeDtypeStruct((N, D), jnp.float32),
        mesh=mesh,
        scratch_types=[
            pltpu.VMEM((_T_PER_W,), jnp.int32),
            pltpu.VMEM((_T_PER_W,), jnp.int32),
            pltpu.VMEM((_T_PER_W, D), jnp.float32),
            pltpu.VMEM((_T_PER_W, D), jnp.float32),
            pltpu.VMEM((_T_PER_W, D), jnp.float32),
            pltpu.SemaphoreType.DMA,
        ],
    )
    def combine_k(x_hbm, og_hbm, s0_hbm, s1_hbm, out_hbm,
                  idx0, idx1, g0, g1, xv, sem):
        wid = lax.axis_index("s") * NC + lax.axis_index("c")
        base = wid * _T_PER_W
        pltpu.sync_copy(s0_hbm.at[pl.ds(base, _T_PER_W)], idx0)
        pltpu.sync_copy(s1_hbm.at[pl.ds(base, _T_PER_W)], idx1)
        pltpu.async_copy(og_hbm.at[idx0], g0, sem).wait()
        pltpu.async_copy(og_hbm.at[idx1], g1, sem).wait()
        pltpu.sync_copy(x_hbm.at[pl.ds(base, _T_PER_W)], xv)

        def row(r, carry):
            for c in range(D // 16):
                sl = pl.ds(c * 16, 16)
                g0[r, sl] = g0[r, sl] + g1[r, sl] + xv[r, sl]
            return carry

        lax.fori_loop(0, _T_PER_W, row, 0)
        pltpu.sync_copy(g0, out_hbm.at[pl.ds(base, _T_PER_W)])

    return combine_k


def _sc_combine(x2, og, s0, s1):
    return _sc_combine_kernel()(x2, og, s0, s1)


def kernel(x, norm_scale, gate_w, gate_b, mlp1_w, mlp1_b, mlp2_w, mlp2_b):
    batch, n_tokens, d = x.shape
    x2 = x.reshape(N, D)
    scale2 = norm_scale.reshape(1, D)
    gwp = jnp.zeros((EP, D), jnp.float32).at[:E].set(gate_w)
    gbp = jnp.zeros((1, EP), jnp.float32).at[0, :E].set(gate_b)

    t, be_rep, act_rep, s0_rep, s1_rep, w0_rep, w1_rep = _route_call(
        x2, scale2, gwp, gbp)

    be = be_rep[:, 0]
    act = act_rep[:, 0]
    s0 = s0_rep[:, 0]
    s1 = s1_rep[:, 0]

    og = _moe_call(be, act, t, s0.reshape(1, N), s1.reshape(1, N),
                   w0_rep[:, 0:1], w1_rep[:, 0:1], mlp1_w,
                   mlp1_b.reshape(E, 2 * FF, 1), mlp2_w,
                   mlp2_b.reshape(E, 1, D))
    out = _sc_combine(x2, og, s0, s1)
    return out.reshape(batch, n_tokens, d)


# speedup vs baseline: 1.2526x; 1.2526x over previous
import functools

import jax
import jax.numpy as jnp
from jax import lax
from jax.experimental import pallas as pl
from jax.experimental.pallas import tpu as pltpu
from jax.experimental.pallas import tpu_sc as plsc

N = 512
D = 1024
E = 8
EP = 128
FF = 2048
BLK = 128
NBLK = 16
NSLOT = NBLK * BLK
EPS = 1e-5
ALPHA = 1.702
LIMIT = 7.0
NC, NS = 2, 16
NW = NC * NS


def _route_body(x_ref, scale_ref, gw_ref, gb_ref,
                t_ref, be_ref, act_ref, s0_ref, s1_ref, w0_ref, w1o_ref):
    x = x_ref[...]
    t = x * lax.rsqrt(jnp.mean(x * x, axis=1, keepdims=True) + EPS)
    t = t * scale_ref[...]
    t_ref[...] = t.astype(jnp.bfloat16)

    lane = lax.broadcasted_iota(jnp.int32, (N, EP), 1)
    g = jax.lax.dot_general(t, gw_ref[...], (((1,), (1,)), ((), ())),
                            preferred_element_type=jnp.float32)
    g = g + gb_ref[...]
    g = jnp.where(lane < E, g, -jnp.inf)

    m1 = jnp.max(g, axis=1, keepdims=True)
    i1 = jnp.min(jnp.where(g == m1, lane, EP), axis=1, keepdims=True)
    g2 = jnp.where(lane == i1, -jnp.inf, g)
    m2 = jnp.max(g2, axis=1, keepdims=True)
    i2 = jnp.min(jnp.where(g2 == m2, lane, EP), axis=1, keepdims=True)
    e2 = jnp.exp(m2 - m1)
    w1c = 1.0 / (1.0 + e2)
    w2c = e2 / (1.0 + e2)

    oh1 = (lane == i1).astype(jnp.float32)
    oh2 = (lane == i2).astype(jnp.float32)
    oh = oh1 + oh2
    counts = jnp.sum(oh, axis=0, keepdims=True)

    r_iota = lax.broadcasted_iota(jnp.int32, (N, N), 0)
    c_iota = lax.broadcasted_iota(jnp.int32, (N, N), 1)
    tril = (r_iota >= c_iota).astype(jnp.float32)
    ranks = jax.lax.dot_general(tril, oh, (((1,), (0,)), ((), ())),
                                preferred_element_type=jnp.float32)
    rank1 = jnp.sum(ranks * oh1, axis=1, keepdims=True)
    rank2 = jnp.sum(ranks * oh2, axis=1, keepdims=True)

    nb = jnp.ceil(counts * (1.0 / BLK))
    lane_r = lax.broadcasted_iota(jnp.int32, (EP, EP), 0)
    lane_c = lax.broadcasted_iota(jnp.int32, (EP, EP), 1)
    ut = (lane_r < lane_c).astype(jnp.float32)
    bs = jax.lax.dot_general(nb, ut, (((1,), (0,)), ((), ())),
                             preferred_element_type=jnp.float32)
    nbt = jnp.sum(nb, axis=1, keepdims=True)
    slot_base = bs * BLK

    sb1 = jnp.sum(slot_base * oh1, axis=1, keepdims=True)
    sb2 = jnp.sum(slot_base * oh2, axis=1, keepdims=True)
    slot0 = sb1 + rank1 - 1.0
    slot1 = sb2 + rank2 - 1.0
    s0_ref[...] = jnp.broadcast_to(slot0, (N, EP)).astype(jnp.int32)
    s1_ref[...] = jnp.broadcast_to(slot1, (N, EP)).astype(jnp.int32)
    w0_ref[...] = jnp.broadcast_to(w1c, (N, EP))
    w1o_ref[...] = jnp.broadcast_to(w2c, (N, EP))

    j_col = lax.broadcasted_iota(jnp.int32, (NBLK, EP), 0).astype(jnp.float32)
    jj = jnp.minimum(j_col, nbt - 1.0)
    e_lane = lax.broadcasted_iota(jnp.int32, (NBLK, EP), 1).astype(jnp.float32)
    inb = jnp.where((jj >= bs) & (jj < bs + nb), 1.0, 0.0)
    be = jnp.sum(inb * e_lane, axis=1, keepdims=True)
    be_ref[...] = jnp.broadcast_to(be, (NBLK, EP)).astype(jnp.int32)
    act_ref[...] = (j_col < nbt).astype(jnp.int32)


def _route_call(x2, scale2, gwp, gbp):
    return pl.pallas_call(
        _route_body,
        out_shape=(
            jax.ShapeDtypeStruct((N, D), jnp.bfloat16),
            jax.ShapeDtypeStruct((NBLK, EP), jnp.int32),
            jax.ShapeDtypeStruct((NBLK, EP), jnp.int32),
            jax.ShapeDtypeStruct((N, EP), jnp.int32),
            jax.ShapeDtypeStruct((N, EP), jnp.int32),
            jax.ShapeDtypeStruct((N, EP), jnp.float32),
            jax.ShapeDtypeStruct((N, EP), jnp.float32),
        ),
    )(x2, scale2, gwp, gbp)


def _moe_body(be_ref, act_ref, t_ref, s0_ref, s1_ref, wc0_ref, wc1_ref,
              w1_ref, b1_ref, w2_ref, b2_ref, og_ref):
    j = pl.program_id(0)

    @pl.when(act_ref[j] == 1)
    def _():
        sid = lax.broadcasted_iota(jnp.int32, (BLK, N), 0) + j * BLK
        oh0 = s0_ref[...] == sid
        oh1 = s1_ref[...] == sid
        oht = (oh0 | oh1).astype(jnp.bfloat16)
        rb = jax.lax.dot_general(oht, t_ref[...], (((1,), (0,)), ((), ())),
                                 preferred_element_type=jnp.float32
                                 ).astype(jnp.bfloat16)
        swc = (jax.lax.dot_general(oh0.astype(jnp.float32), wc0_ref[...],
                                   (((1,), (0,)), ((), ())),
                                   preferred_element_type=jnp.float32) +
               jax.lax.dot_general(oh1.astype(jnp.float32), wc1_ref[...],
                                   (((1,), (0,)), ((), ())),
                                   preferred_element_type=jnp.float32))
        w1b = w1_ref[0].astype(jnp.bfloat16)
        ht = jax.lax.dot_general(w1b, rb, (((1,), (1,)), ((), ())),
                                 preferred_element_type=jnp.float32)
        ht = ht + b1_ref[0]
        h2 = ht.reshape(FF, 2 * BLK)
        xg = jnp.minimum(h2[:, :BLK], LIMIT)
        xl = jnp.clip(h2[:, BLK:], -LIMIT, LIMIT)
        a = xg * (1.0 / (1.0 + jnp.exp(-ALPHA * xg))) * (xl + 1.0)
        at = jnp.transpose(a) * swc
        w2b = w2_ref[0].astype(jnp.bfloat16)
        o = jax.lax.dot_general(at.astype(jnp.bfloat16), w2b,
                                (((1,), (1,)), ((), ())),
                                preferred_element_type=jnp.float32)
        og_ref[...] = o + b2_ref[0] * swc


def _moe_call(be, act, t_bf, s0row, s1row, wc0, wc1, w1r, b1p, w2, b2r):
    grid_spec = pltpu.PrefetchScalarGridSpec(
        num_scalar_prefetch=2,
        grid=(NBLK,),
        in_specs=[
            pl.BlockSpec((N, D), lambda j, be, act: (0, 0)),
            pl.BlockSpec((1, N), lambda j, be, act: (0, 0)),
            pl.BlockSpec((1, N), lambda j, be, act: (0, 0)),
            pl.BlockSpec((N, 1), lambda j, be, act: (0, 0)),
            pl.BlockSpec((N, 1), lambda j, be, act: (0, 0)),
            pl.BlockSpec((1, 2 * FF, D), lambda j, be, act: (be[j], 0, 0)),
            pl.BlockSpec((1, 2 * FF, 1), lambda j, be, act: (be[j], 0, 0)),
            pl.BlockSpec((1, D, FF), lambda j, be, act: (be[j], 0, 0)),
            pl.BlockSpec((1, 1, D), lambda j, be, act: (be[j], 0, 0)),
        ],
        out_specs=pl.BlockSpec((BLK, D), lambda j, be, act: (j, 0)),
    )
    return pl.pallas_call(
        _moe_body,
        grid_spec=grid_spec,
        out_shape=jax.ShapeDtypeStruct((NSLOT, D), jnp.float32),
    )(be, act, t_bf, s0row, s1row, wc0, wc1, w1r, b1p, w2, b2r)


_T_PER_W = N // NW
@functools.lru_cache(maxsize=None)
def _sc_combine_kernel():
    mesh = plsc.VectorSubcoreMesh(core_axis_name="c", subcore_axis_name="s")

    @functools.partial(
        pl.kernel,
        out_type=jax.ShapeDtypeStruct((N, D), jnp.float32),
        mesh=mesh,
        scratch_types=[
            pltpu.VMEM((_T_PER_W,), jnp.int32),
            pltpu.VMEM((_T_PER_W,), jnp.int32),
            pltpu.VMEM((_T_PER_W, D), jnp.float32),
            pltpu.VMEM((_T_PER_W, D), jnp.float32),
            pltpu.VMEM((_T_PER_W, D), jnp.float32),
            pltpu.SemaphoreType.DMA,
        ],
    )
    def combine_k(x_hbm, og_hbm, s0_hbm, s1_hbm, out_hbm,
                  idx0, idx1, g0, g1, xv, sem):
        wid = lax.axis_index("s") * NC + lax.axis_index("c")
        base = wid * _T_PER_W
        pltpu.sync_copy(s0_hbm.at[pl.ds(base, _T_PER_W)], idx0)
        pltpu.sync_copy(s1_hbm.at[pl.ds(base, _T_PER_W)], idx1)
        pltpu.async_copy(og_hbm.at[idx0], g0, sem).wait()
        pltpu.async_copy(og_hbm.at[idx1], g1, sem).wait()
        pltpu.sync_copy(x_hbm.at[pl.ds(base, _T_PER_W)], xv)

        def row(r, carry):
            for c in range(D // 16):
                sl = pl.ds(c * 16, 16)
                g0[r, sl] = g0[r, sl] + g1[r, sl] + xv[r, sl]
            return carry

        lax.fori_loop(0, _T_PER_W, row, 0)
        pltpu.sync_copy(g0, out_hbm.at[pl.ds(base, _T_PER_W)])

    return combine_k


def _sc_combine(x2, og, s0, s1):
    return _sc_combine_kernel()(x2, og, s0, s1)


def kernel(x, norm_scale, gate_w, gate_b, mlp1_w, mlp1_b, mlp2_w, mlp2_b):
    batch, n_tokens, d = x.shape
    x2 = x.reshape(N, D)
    scale2 = norm_scale.reshape(1, D)
    gwp = jnp.zeros((EP, D), jnp.float32).at[:E].set(gate_w)
    gbp = jnp.zeros((1, EP), jnp.float32).at[0, :E].set(gate_b)

    t, be_rep, act_rep, s0_rep, s1_rep, w0_rep, w1_rep = _route_call(
        x2, scale2, gwp, gbp)

    be = be_rep[:, 0]
    act = act_rep[:, 0]
    s0 = s0_rep[:, 0]
    s1 = s1_rep[:, 0]

    og = _moe_call(be, act, t, s0.reshape(1, N), s1.reshape(1, N),
                   w0_rep[:, 0:1], w1_rep[:, 0:1], mlp1_w,
                   mlp1_b.reshape(E, 2 * FF, 1), mlp2_w,
                   mlp2_b.reshape(E, 1, D))
    out = _sc_combine(x2, og, s0, s1)
    return out.reshape(batch, n_tokens, d)
